# baseline (speedup 1.0000x reference)
import functools

import jax
import jax.numpy as jnp
from jax import lax
from jax.experimental import pallas as pl
from jax.experimental.pallas import tpu as pltpu
from jax.experimental.pallas import tpu_sc as plsc

_B = 16384
_F = 26
_V = 100000
_D = 16
_H = 128
_EPS = 1e-5

_NG = _F - 2
_ROWS = _B * _NG
_NC, _NS = 2, 16
_NW = _NC * _NS
_RPW = _ROWS // _NW
_CH = 2048
_NCH = _RPW // _CH

_TB = 1024
_NT = _B // _TB


def _sc_gather_body(tbl_hbm, idx_hbm, out_hbm, idx_v, rows_v, sem):
    wid = lax.axis_index("s") * _NC + lax.axis_index("c")
    pltpu.sync_copy(idx_hbm.at[wid], idx_v)

    def body(c, carry):
        base = wid * _RPW + c * _CH
        pltpu.async_copy(
            tbl_hbm.at[idx_v.at[pl.ds(c * _CH, _CH)]], rows_v, sem
        ).wait()
        pltpu.sync_copy(rows_v, out_hbm.at[pl.ds(base, _CH)])
        return carry

    lax.fori_loop(0, _NCH, body, 0)


def _gather_rows(tables_flat, idx_mat):
    mesh = plsc.VectorSubcoreMesh(core_axis_name="c", subcore_axis_name="s")
    k = functools.partial(
        pl.kernel,
        mesh=mesh,
        compiler_params=pltpu.CompilerParams(use_tc_tiling_on_sc=False),
        out_type=jax.ShapeDtypeStruct((_ROWS, _D), jnp.float32),
        scratch_types=[
            pltpu.VMEM((_RPW,), jnp.int32),
            pltpu.VMEM((_CH, _D), jnp.float32),
            pltpu.SemaphoreType.DMA,
        ],
    )(_sc_gather_body)
    return k(tables_flat, idx_mat)


def _mlp1_body(emb_ref, xf_ref, t01_ref, w1a_ref, w1g_ref, b1_ref,
               h1_ref, st_ref):
    i = pl.program_id(0)
    a0 = xf_ref[:, 0:1] * t01_ref[0:1, :]
    a1 = xf_ref[:, 1:2] * t01_ref[1:2, :]
    h = jnp.dot(emb_ref[...], w1g_ref[...], preferred_element_type=jnp.float32)
    h = h + jnp.dot(a0, w1a_ref[0:16, :], preferred_element_type=jnp.float32)
    h = h + jnp.dot(a1, w1a_ref[16:32, :], preferred_element_type=jnp.float32)
    h = h + b1_ref[0:1, :]
    h1_ref[...] = h

    @pl.when(i == 0)
    def _():
        st_ref[...] = jnp.zeros_like(st_ref)

    st_ref[0:1, :] += jnp.sum(h, axis=0, keepdims=True)
    st_ref[1:2, :] += jnp.sum(h * h, axis=0, keepdims=True)


def _mlp2_body(h1_ref, st1_ref, gb1_ref, w2_ref, b2_ref, h2_ref, st_ref):
    i = pl.program_id(0)
    m = st1_ref[0:1, :] * (1.0 / _B)
    v = st1_ref[1:2, :] * (1.0 / _B) - m * m
    r = lax.rsqrt(v + _EPS)
    h1n = (h1_ref[...] - m) * (r * gb1_ref[0:1, :]) + gb1_ref[1:2, :]
    h2 = jnp.dot(h1n, w2_ref[...], preferred_element_type=jnp.float32)
    h2 = h2 + b2_ref[0:1, :]
    h2_ref[...] = h2

    @pl.when(i == 0)
    def _():
        st_ref[...] = jnp.zeros_like(st_ref)

    st_ref[0:1, :] += jnp.sum(h2, axis=0, keepdims=True)
    st_ref[1:2, :] += jnp.sum(h2 * h2, axis=0, keepdims=True)


def _head_body(h2_ref, st2_ref, gb2_ref, wp_ref, bp_ref,
               ed_ref, lg_ref, pr_ref):
    m = st2_ref[0:1, :] * (1.0 / _B)
    v = st2_ref[1:2, :] * (1.0 / _B) - m * m
    r = lax.rsqrt(v + _EPS)
    h2n = (h2_ref[...] - m) * (r * gb2_ref[0:1, :]) + gb2_ref[1:2, :]
    ed_ref[...] = h2n
    lg = jnp.dot(h2n, wp_ref[...], preferred_element_type=jnp.float32)
    lg = lg + bp_ref[0:1, :]
    l0 = lg[:, 0:1]
    l1 = lg[:, 1:2]
    mx = jnp.maximum(l0, l1)
    e0 = jnp.exp(l0 - mx)
    e1 = jnp.exp(l1 - mx)
    s = e0 + e1
    lg_ref[...] = jnp.concatenate([l0, l1], axis=1)
    pr_ref[...] = jnp.concatenate([e0 / s, e1 / s], axis=1)


def kernel(x, tables, W1, b1, g1, be1, W2, b2, g2, be2, Wp, bp):
    f32 = jnp.float32
    x0 = x[0]
    xf = x0[:, :2].astype(f32)
    offs = (jnp.arange(2, _F, dtype=jnp.int32) * _V)[None, :]
    idx = (x0[:, 2:] + offs).reshape(_NW, _RPW)
    tables_flat = tables.reshape(_F * _V, _D)

    rows = _gather_rows(tables_flat, idx)
    emb_g = rows.reshape(_B, _NG * _D)

    t01 = jnp.pad(tables[:2, 0, :], ((0, 6), (0, 0)))
    W1a = W1[:2 * _D, :]
    W1g = W1[2 * _D:, :]
    b1p = jnp.pad(b1[None, :], ((0, 7), (0, 0)))
    gb1 = jnp.pad(jnp.stack([g1, be1]), ((0, 6), (0, 0)))
    b2p = jnp.pad(b2[None, :], ((0, 7), (0, 0)))
    gb2 = jnp.pad(jnp.stack([g2, be2]), ((0, 6), (0, 0)))
    wpp = jnp.pad(Wp, ((0, 0), (0, 6)))
    bpp = jnp.pad(bp[None, :], ((0, 7), (0, 6)))

    full = lambda shape: pl.BlockSpec(shape, lambda i: (0, 0))
    tile = lambda w: pl.BlockSpec((_TB, w), lambda i: (i, 0))

    h1, st1 = pl.pallas_call(
        _mlp1_body,
        grid=(_NT,),
        in_specs=[tile(_NG * _D), tile(2), full((8, 16)), full((32, _H)),
                  full((_NG * _D, _H)), full((8, _H))],
        out_specs=[tile(_H), full((8, _H))],
        out_shape=[jax.ShapeDtypeStruct((_B, _H), f32),
                   jax.ShapeDtypeStruct((8, _H), f32)],
    )(emb_g, xf, t01, W1a, W1g, b1p)

    h2, st2 = pl.pallas_call(
        _mlp2_body,
        grid=(_NT,),
        in_specs=[tile(_H), full((8, _H)), full((8, _H)), full((_H, _H)),
                  full((8, _H))],
        out_specs=[tile(_H), full((8, _H))],
        out_shape=[jax.ShapeDtypeStruct((_B, _H), f32),
                   jax.ShapeDtypeStruct((8, _H), f32)],
    )(h1, st1, gb1, W2, b2p)

    emb_deep, logit, pred = pl.pallas_call(
        _head_body,
        grid=(_NT,),
        in_specs=[tile(_H), full((8, _H)), full((8, _H)), full((_H, 8)),
                  full((8, 8))],
        out_specs=[tile(_H), tile(2), tile(2)],
        out_shape=[jax.ShapeDtypeStruct((_B, _H), f32),
                   jax.ShapeDtypeStruct((_B, 2), f32),
                   jax.ShapeDtypeStruct((_B, 2), f32)],
    )(h2, st2, gb2, wpp, bpp)

    return (emb_deep, logit, pred)


# speedup vs baseline: 1.9999x; 1.9999x over previous
import functools

import jax
import jax.numpy as jnp
from jax import lax
from jax.experimental import pallas as pl
from jax.experimental.pallas import tpu as pltpu
from jax.experimental.pallas import tpu_sc as plsc

_B = 16384
_F = 26
_V = 100000
_D = 16
_H = 128
_EPS = 1e-5

_NG = _F - 2
_ROWS = _B * _NG
_NC, _NS = 2, 16
_NW = _NC * _NS
_RPW = _ROWS // _NW
_CH = 2048
_NCH = _RPW // _CH

_TB = 1024
_NT = _B // _TB


_VMAIN = (_V // 128) * 128
_AUXB = _F * _VMAIN
_DVLEN = _F * _D * _VMAIN

_K1W = 1408
_K1PF = _VMAIN // _K1W
_K1N = _K1PF * _F
_K1PW = (_K1N + _NW - 1) // _NW

_K2W = 2272
_K2PF = _VMAIN // _K2W
_K2N = _K2PF * _F
_K2PW = (_K2N + _NW - 1) // _NW


def _sc_untile_body(tbl_hbm, out_hbm, staged):
    wid = lax.axis_index("s") * _NC + lax.axis_index("c")

    def body(j, carry):
        g = wid + _NW * j

        @pl.when(g < _K1N)
        def _():
            f = g // _K1PF
            v0 = (g % _K1PF) * _K1W
            pltpu.sync_copy(tbl_hbm.at[f, :, pl.ds(v0, _K1W)], staged)
            for d in range(_D):
                pltpu.sync_copy(
                    staged.at[d, :],
                    out_hbm.at[pl.ds((f * _D + d) * _VMAIN + v0, _K1W)])

        return carry

    lax.fori_loop(0, _K1PW, body, 0)


def _untile_tables(tables_t):
    mesh = plsc.VectorSubcoreMesh(core_axis_name="c", subcore_axis_name="s")
    k = functools.partial(
        pl.kernel,
        mesh=mesh,
        compiler_params=pltpu.CompilerParams(use_tc_tiling_on_sc=True),
        out_type=jax.ShapeDtypeStruct((_DVLEN,), jnp.float32),
        scratch_types=[pltpu.VMEM((_D, _K1W), jnp.float32)],
    )(_sc_untile_body)
    return k(tables_t)


def _sc_transpose_body(dv_hbm, out_hbm, staged, obuf):
    wid = lax.axis_index("s") * _NC + lax.axis_index("c")
    bases = [lax.iota(jnp.int32, 16) * _D + d for d in range(_D)]

    def body(i, carry):
        c = wid + _NW * i

        @pl.when(c < _K2N)
        def _():
            f = c // _K2PF
            v0 = (c % _K2PF) * _K2W
            for d in range(_D):
                pltpu.sync_copy(
                    dv_hbm.at[pl.ds((f * _D + d) * _VMAIN + v0, _K2W)],
                    staged.at[d, :])

            def jbody(j, carry2):
                for d in range(_D):
                    xv = staged[d, pl.ds(j * 16, 16)]
                    plsc.store_scatter(obuf, [bases[d] + j * (16 * _D)], xv)
                return carry2

            lax.fori_loop(0, _K2W // 16, jbody, 0)
            pltpu.sync_copy(
                obuf,
                out_hbm.at[pl.ds((f * _VMAIN + v0) * _D, _K2W * _D)])

        return carry

    lax.fori_loop(0, _K2PW, body, 0)


def _transpose_tables(t_dv):
    mesh = plsc.VectorSubcoreMesh(core_axis_name="c", subcore_axis_name="s")
    k = functools.partial(
        pl.kernel,
        mesh=mesh,
        compiler_params=pltpu.CompilerParams(use_tc_tiling_on_sc=False,
                                             needs_layout_passes=False),
        out_type=jax.ShapeDtypeStruct((_F * _V * _D,), jnp.float32),
        scratch_types=[
            pltpu.VMEM((_D, _K2W), jnp.float32),
            pltpu.VMEM((_K2W * _D,), jnp.float32),
        ],
    )(_sc_transpose_body)
    return k(t_dv)


def _sc_fused_relayout_body(tbl_hbm, out_hbm, staged, obuf):
    wid = lax.axis_index("s") * _NC + lax.axis_index("c")
    bases = [lax.iota(jnp.int32, 16) * _D + d for d in range(_D)]

    def body(i, carry):
        c = wid + _NW * i

        @pl.when(c < _K1N)
        def _():
            f = c // _K1PF
            v0 = (c % _K1PF) * _K1W
            pltpu.sync_copy(tbl_hbm.at[f, :, pl.ds(v0, _K1W)], staged)

            def jbody(j, carry2):
                for d in range(_D):
                    xv = staged[d, pl.ds(j * 16, 16)]
                    plsc.store_scatter(obuf, [bases[d] + j * (16 * _D)], xv)
                return carry2

            lax.fori_loop(0, _K1W // 16, jbody, 0)
            pltpu.sync_copy(
                obuf,
                out_hbm.at[pl.ds((f * _VMAIN + v0) * _D, _K1W * _D)])

        return carry

    lax.fori_loop(0, _K1PW, body, 0)


def _fused_relayout_tables(tables_t):
    mesh = plsc.VectorSubcoreMesh(core_axis_name="c", subcore_axis_name="s")
    k = functools.partial(
        pl.kernel,
        mesh=mesh,
        compiler_params=pltpu.CompilerParams(use_tc_tiling_on_sc=True,
                                             needs_layout_passes=False),
        out_type=jax.ShapeDtypeStruct((_F * _V * _D,), jnp.float32),
        scratch_types=[
            pltpu.VMEM((_D, _K1W), jnp.float32),
            pltpu.VMEM((_K1W * _D,), jnp.float32),
        ],
    )(_sc_fused_relayout_body)
    return k(tables_t)


def _relayout_tables(tables_t):
    return _fused_relayout_tables(tables_t)


def _sc_gather_body(tbl_hbm, idx_hbm, out_hbm, idx_v, rows_v, sem):
    wid = lax.axis_index("s") * _NC + lax.axis_index("c")
    pltpu.sync_copy(idx_hbm.at[wid], idx_v)

    def body(c, carry):
        base = wid * _RPW + c * _CH
        pltpu.async_copy(
            tbl_hbm.at[idx_v.at[pl.ds(c * _CH, _CH)]], rows_v, sem
        ).wait()
        pltpu.sync_copy(rows_v, out_hbm.at[pl.ds(base, _CH)])
        return carry

    lax.fori_loop(0, _NCH, body, 0)


def _gather_rows(tables_flat, idx_mat):
    mesh = plsc.VectorSubcoreMesh(core_axis_name="c", subcore_axis_name="s")
    k = functools.partial(
        pl.kernel,
        mesh=mesh,
        compiler_params=pltpu.CompilerParams(use_tc_tiling_on_sc=False),
        out_type=jax.ShapeDtypeStruct((_ROWS, _D), jnp.float32),
        scratch_types=[
            pltpu.VMEM((_RPW,), jnp.int32),
            pltpu.VMEM((_CH, _D), jnp.float32),
            pltpu.SemaphoreType.DMA,
        ],
    )(_sc_gather_body)
    return k(tables_flat, idx_mat)


def _mlp1_body(emb_ref, xf_ref, t01_ref, w1a_ref, w1g_ref, b1_ref,
               h1_ref, st_ref):
    i = pl.program_id(0)
    a0 = xf_ref[:, 0:1] * t01_ref[0:1, :]
    a1 = xf_ref[:, 1:2] * t01_ref[1:2, :]
    h = jnp.dot(emb_ref[...], w1g_ref[...], preferred_element_type=jnp.float32)
    h = h + jnp.dot(a0, w1a_ref[0:16, :], preferred_element_type=jnp.float32)
    h = h + jnp.dot(a1, w1a_ref[16:32, :], preferred_element_type=jnp.float32)
    h = h + b1_ref[0:1, :]
    h1_ref[...] = h

    @pl.when(i == 0)
    def _():
        st_ref[...] = jnp.zeros_like(st_ref)

    st_ref[0:1, :] += jnp.sum(h, axis=0, keepdims=True)
    st_ref[1:2, :] += jnp.sum(h * h, axis=0, keepdims=True)


def _mlp2_body(h1_ref, st1_ref, gb1_ref, w2_ref, b2_ref, h2_ref, st_ref):
    i = pl.program_id(0)
    m = st1_ref[0:1, :] * (1.0 / _B)
    v = st1_ref[1:2, :] * (1.0 / _B) - m * m
    r = lax.rsqrt(v + _EPS)
    h1n = (h1_ref[...] - m) * (r * gb1_ref[0:1, :]) + gb1_ref[1:2, :]
    h2 = jnp.dot(h1n, w2_ref[...], preferred_element_type=jnp.float32)
    h2 = h2 + b2_ref[0:1, :]
    h2_ref[...] = h2

    @pl.when(i == 0)
    def _():
        st_ref[...] = jnp.zeros_like(st_ref)

    st_ref[0:1, :] += jnp.sum(h2, axis=0, keepdims=True)
    st_ref[1:2, :] += jnp.sum(h2 * h2, axis=0, keepdims=True)


def _head_body(h2_ref, st2_ref, gb2_ref, wp_ref, bp_ref,
               ed_ref, lg_ref, pr_ref):
    m = st2_ref[0:1, :] * (1.0 / _B)
    v = st2_ref[1:2, :] * (1.0 / _B) - m * m
    r = lax.rsqrt(v + _EPS)
    h2n = (h2_ref[...] - m) * (r * gb2_ref[0:1, :]) + gb2_ref[1:2, :]
    ed_ref[...] = h2n
    lg = jnp.dot(h2n, wp_ref[...], preferred_element_type=jnp.float32)
    lg = lg + bp_ref[0:1, :]
    l0 = lg[:, 0:1]
    l1 = lg[:, 1:2]
    mx = jnp.maximum(l0, l1)
    e0 = jnp.exp(l0 - mx)
    e1 = jnp.exp(l1 - mx)
    s = e0 + e1
    lg_ref[...] = jnp.concatenate([l0, l1], axis=1)
    pr_ref[...] = jnp.concatenate([e0 / s, e1 / s], axis=1)


def kernel(x, tables, W1, b1, g1, be1, W2, b2, g2, be2, Wp, bp):
    f32 = jnp.float32
    x0 = x[0]
    xf = x0[:, :2].astype(f32)
    v24 = x0[:, 2:]
    f24 = jnp.arange(2, _F, dtype=jnp.int32)[None, :]
    in_aux = v24 >= _VMAIN
    idx = jnp.where(in_aux,
                    _AUXB + f24 * (_V - _VMAIN) + (v24 - _VMAIN),
                    f24 * _VMAIN + v24).reshape(_NW, _RPW)

    tables_t = jnp.transpose(tables, (0, 2, 1))
    tlin = _relayout_tables(tables_t)
    aux = tables[:, _VMAIN:, :].reshape(-1)
    tlin = lax.dynamic_update_slice(tlin, aux, (_AUXB * _D,))
    tables_flat = tlin.reshape(_F * _V, _D)

    rows = _gather_rows(tables_flat, idx)
    emb_g = rows.reshape(_B, _NG * _D)

    t01 = jnp.pad(tables[:2, 0, :], ((0, 6), (0, 0)))
    W1a = W1[:2 * _D, :]
    W1g = W1[2 * _D:, :]
    b1p = jnp.pad(b1[None, :], ((0, 7), (0, 0)))
    gb1 = jnp.pad(jnp.stack([g1, be1]), ((0, 6), (0, 0)))
    b2p = jnp.pad(b2[None, :], ((0, 7), (0, 0)))
    gb2 = jnp.pad(jnp.stack([g2, be2]), ((0, 6), (0, 0)))
    wpp = jnp.pad(Wp, ((0, 0), (0, 6)))
    bpp = jnp.pad(bp[None, :], ((0, 7), (0, 6)))

    full = lambda shape: pl.BlockSpec(shape, lambda i: (0, 0))
    tile = lambda w: pl.BlockSpec((_TB, w), lambda i: (i, 0))

    h1, st1 = pl.pallas_call(
        _mlp1_body,
        grid=(_NT,),
        in_specs=[tile(_NG * _D), tile(2), full((8, 16)), full((32, _H)),
                  full((_NG * _D, _H)), full((8, _H))],
        out_specs=[tile(_H), full((8, _H))],
        out_shape=[jax.ShapeDtypeStruct((_B, _H), f32),
                   jax.ShapeDtypeStruct((8, _H), f32)],
    )(emb_g, xf, t01, W1a, W1g, b1p)

    h2, st2 = pl.pallas_call(
        _mlp2_body,
        grid=(_NT,),
        in_specs=[tile(_H), full((8, _H)), full((8, _H)), full((_H, _H)),
                  full((8, _H))],
        out_specs=[tile(_H), full((8, _H))],
        out_shape=[jax.ShapeDtypeStruct((_B, _H), f32),
                   jax.ShapeDtypeStruct((8, _H), f32)],
    )(h1, st1, gb1, W2, b2p)

    emb_deep, logit, pred = pl.pallas_call(
        _head_body,
        grid=(_NT,),
        in_specs=[tile(_H), full((8, _H)), full((8, _H)), full((_H, 8)),
                  full((8, 8))],
        out_specs=[tile(_H), tile(2), tile(2)],
        out_shape=[jax.ShapeDtypeStruct((_B, _H), f32),
                   jax.ShapeDtypeStruct((_B, 2), f32),
                   jax.ShapeDtypeStruct((_B, 2), f32)],
    )(h2, st2, gb2, wpp, bpp)

    return (emb_deep, logit, pred)


# speedup vs baseline: 2.6822x; 1.3411x over previous
import functools

import jax
import jax.numpy as jnp
from jax import lax
from jax.experimental import pallas as pl
from jax.experimental.pallas import tpu as pltpu
from jax.experimental.pallas import tpu_sc as plsc

_B = 16384
_F = 26
_V = 100000
_D = 16
_H = 128
_EPS = 1e-5

_NG = _F - 2
_ROWS = _B * _NG
_NC, _NS = 2, 16
_NW = _NC * _NS
_RPW = _ROWS // _NW
_CH = 2048
_NCH = _RPW // _CH

_TB = 1024
_NT = _B // _TB


_VMAIN = (_V // 128) * 128
_AUXB = _F * _VMAIN
_DVLEN = _F * _D * _VMAIN

_K1W = 1408
_K1PF = _VMAIN // _K1W
_K1N = _K1PF * _F
_K1PW = (_K1N + _NW - 1) // _NW

_K2W = 2272
_K2PF = _VMAIN // _K2W
_K2N = _K2PF * _F
_K2PW = (_K2N + _NW - 1) // _NW


def _sc_untile_body(tbl_hbm, out_hbm, staged):
    wid = lax.axis_index("s") * _NC + lax.axis_index("c")

    def body(j, carry):
        g = wid + _NW * j

        @pl.when(g < _K1N)
        def _():
            f = g // _K1PF
            v0 = (g % _K1PF) * _K1W
            pltpu.sync_copy(tbl_hbm.at[f, :, pl.ds(v0, _K1W)], staged)
            for d in range(_D):
                pltpu.sync_copy(
                    staged.at[d, :],
                    out_hbm.at[pl.ds((f * _D + d) * _VMAIN + v0, _K1W)])

        return carry

    lax.fori_loop(0, _K1PW, body, 0)


def _untile_tables(tables_t):
    mesh = plsc.VectorSubcoreMesh(core_axis_name="c", subcore_axis_name="s")
    k = functools.partial(
        pl.kernel,
        mesh=mesh,
        compiler_params=pltpu.CompilerParams(use_tc_tiling_on_sc=True),
        out_type=jax.ShapeDtypeStruct((_DVLEN,), jnp.float32),
        scratch_types=[pltpu.VMEM((_D, _K1W), jnp.float32)],
    )(_sc_untile_body)
    return k(tables_t)


def _sc_transpose_body(dv_hbm, out_hbm, staged, obuf):
    wid = lax.axis_index("s") * _NC + lax.axis_index("c")
    bases = [lax.iota(jnp.int32, 16) * _D + d for d in range(_D)]

    def body(i, carry):
        c = wid + _NW * i

        @pl.when(c < _K2N)
        def _():
            f = c // _K2PF
            v0 = (c % _K2PF) * _K2W
            for d in range(_D):
                pltpu.sync_copy(
                    dv_hbm.at[pl.ds((f * _D + d) * _VMAIN + v0, _K2W)],
                    staged.at[d, :])

            def jbody(j, carry2):
                for d in range(_D):
                    xv = staged[d, pl.ds(j * 16, 16)]
                    plsc.store_scatter(obuf, [bases[d] + j * (16 * _D)], xv)
                return carry2

            lax.fori_loop(0, _K2W // 16, jbody, 0)
            pltpu.sync_copy(
                obuf,
                out_hbm.at[pl.ds((f * _VMAIN + v0) * _D, _K2W * _D)])

        return carry

    lax.fori_loop(0, _K2PW, body, 0)


def _transpose_tables(t_dv):
    mesh = plsc.VectorSubcoreMesh(core_axis_name="c", subcore_axis_name="s")
    k = functools.partial(
        pl.kernel,
        mesh=mesh,
        compiler_params=pltpu.CompilerParams(use_tc_tiling_on_sc=False,
                                             needs_layout_passes=False),
        out_type=jax.ShapeDtypeStruct((_F * _V * _D,), jnp.float32),
        scratch_types=[
            pltpu.VMEM((_D, _K2W), jnp.float32),
            pltpu.VMEM((_K2W * _D,), jnp.float32),
        ],
    )(_sc_transpose_body)
    return k(t_dv)


def _sc_fused_relayout_body(tbl_hbm, out_hbm, st0, st1, ob0, ob1,
                            is0, is1, os0, os1):
    wid = lax.axis_index("s") * _NC + lax.axis_index("c")
    bases = [lax.iota(jnp.int32, 16) * _D + d for d in range(_D)]
    staged = (st0, st1)
    obuf = (ob0, ob1)
    isem = (is0, is1)
    osem = (os0, os1)

    def issue_in(c, b):
        f = c // _K1PF
        v0 = (c % _K1PF) * _K1W
        pltpu.async_copy(tbl_hbm.at[f, :, pl.ds(v0, _K1W)], staged[b], isem[b])

    issue_in(wid, 0)

    def body(t, carry):
        for b in range(2):
            i = t * 2 + b
            c = wid + _NW * i
            cn = c + _NW

            @pl.when(c < _K1N)
            def _():
                pltpu.make_async_copy(
                    tbl_hbm.at[0, :, pl.ds(0, _K1W)], staged[b], isem[b]
                ).wait()

            @pl.when(cn < _K1N)
            def _():
                issue_in(cn, 1 - b)

            @pl.when((i >= 2) & (c < _K1N))
            def _():
                pltpu.make_async_copy(
                    ob0, out_hbm.at[pl.ds(0, _K1W * _D)], osem[b]
                ).wait()

            @pl.when(c < _K1N)
            def _():
                f = c // _K1PF
                v0 = (c % _K1PF) * _K1W

                def jbody(j, carry2):
                    for d in range(_D):
                        xv = staged[b][d, pl.ds(j * 16, 16)]
                        plsc.store_scatter(
                            obuf[b], [bases[d] + j * (16 * _D)], xv)
                    return carry2

                lax.fori_loop(0, _K1W // 16, jbody, 0)
                pltpu.async_copy(
                    obuf[b],
                    out_hbm.at[pl.ds((f * _VMAIN + v0) * _D, _K1W * _D)],
                    osem[b])

        return carry

    lax.fori_loop(0, (_K1PW + 1) // 2, body, 0)

    for b in range(2):
        pltpu.make_async_copy(
            ob0, out_hbm.at[pl.ds(0, _K1W * _D)], osem[b]
        ).wait()


def _fused_relayout_tables(tables_t):
    mesh = plsc.VectorSubcoreMesh(core_axis_name="c", subcore_axis_name="s")
    k = functools.partial(
        pl.kernel,
        mesh=mesh,
        compiler_params=pltpu.CompilerParams(use_tc_tiling_on_sc=True,
                                             needs_layout_passes=False),
        out_type=jax.ShapeDtypeStruct((_F * _V * _D,), jnp.float32),
        scratch_types=[
            pltpu.VMEM((_D, _K1W), jnp.float32),
            pltpu.VMEM((_D, _K1W), jnp.float32),
            pltpu.VMEM((_K1W * _D,), jnp.float32),
            pltpu.VMEM((_K1W * _D,), jnp.float32),
            pltpu.SemaphoreType.DMA,
            pltpu.SemaphoreType.DMA,
            pltpu.SemaphoreType.DMA,
            pltpu.SemaphoreType.DMA,
        ],
    )(_sc_fused_relayout_body)
    return k(tables_t)


def _relayout_tables(tables_t):
    return _fused_relayout_tables(tables_t)


def _sc_gather_body(tbl_hbm, idx_hbm, out_hbm, idx_v, rows_v, sem):
    wid = lax.axis_index("s") * _NC + lax.axis_index("c")
    pltpu.sync_copy(idx_hbm.at[wid], idx_v)

    def body(c, carry):
        base = wid * _RPW + c * _CH
        pltpu.async_copy(
            tbl_hbm.at[idx_v.at[pl.ds(c * _CH, _CH)]], rows_v, sem
        ).wait()
        pltpu.sync_copy(rows_v, out_hbm.at[pl.ds(base, _CH)])
        return carry

    lax.fori_loop(0, _NCH, body, 0)


def _gather_rows(tables_flat, idx_mat):
    mesh = plsc.VectorSubcoreMesh(core_axis_name="c", subcore_axis_name="s")
    k = functools.partial(
        pl.kernel,
        mesh=mesh,
        compiler_params=pltpu.CompilerParams(use_tc_tiling_on_sc=False),
        out_type=jax.ShapeDtypeStruct((_ROWS, _D), jnp.float32),
        scratch_types=[
            pltpu.VMEM((_RPW,), jnp.int32),
            pltpu.VMEM((_CH, _D), jnp.float32),
            pltpu.SemaphoreType.DMA,
        ],
    )(_sc_gather_body)
    return k(tables_flat, idx_mat)


def _mlp1_body(emb_ref, xf_ref, t01_ref, w1a_ref, w1g_ref, b1_ref,
               h1_ref, st_ref):
    i = pl.program_id(0)
    a0 = xf_ref[:, 0:1] * t01_ref[0:1, :]
    a1 = xf_ref[:, 1:2] * t01_ref[1:2, :]
    h = jnp.dot(emb_ref[...], w1g_ref[...], preferred_element_type=jnp.float32)
    h = h + jnp.dot(a0, w1a_ref[0:16, :], preferred_element_type=jnp.float32)
    h = h + jnp.dot(a1, w1a_ref[16:32, :], preferred_element_type=jnp.float32)
    h = h + b1_ref[0:1, :]
    h1_ref[...] = h

    @pl.when(i == 0)
    def _():
        st_ref[...] = jnp.zeros_like(st_ref)

    st_ref[0:1, :] += jnp.sum(h, axis=0, keepdims=True)
    st_ref[1:2, :] += jnp.sum(h * h, axis=0, keepdims=True)


def _mlp2_body(h1_ref, st1_ref, gb1_ref, w2_ref, b2_ref, h2_ref, st_ref):
    i = pl.program_id(0)
    m = st1_ref[0:1, :] * (1.0 / _B)
    v = st1_ref[1:2, :] * (1.0 / _B) - m * m
    r = lax.rsqrt(v + _EPS)
    h1n = (h1_ref[...] - m) * (r * gb1_ref[0:1, :]) + gb1_ref[1:2, :]
    h2 = jnp.dot(h1n, w2_ref[...], preferred_element_type=jnp.float32)
    h2 = h2 + b2_ref[0:1, :]
    h2_ref[...] = h2

    @pl.when(i == 0)
    def _():
        st_ref[...] = jnp.zeros_like(st_ref)

    st_ref[0:1, :] += jnp.sum(h2, axis=0, keepdims=True)
    st_ref[1:2, :] += jnp.sum(h2 * h2, axis=0, keepdims=True)


def _head_body(h2_ref, st2_ref, gb2_ref, wp_ref, bp_ref,
               ed_ref, lg_ref, pr_ref):
    m = st2_ref[0:1, :] * (1.0 / _B)
    v = st2_ref[1:2, :] * (1.0 / _B) - m * m
    r = lax.rsqrt(v + _EPS)
    h2n = (h2_ref[...] - m) * (r * gb2_ref[0:1, :]) + gb2_ref[1:2, :]
    ed_ref[...] = h2n
    lg = jnp.dot(h2n, wp_ref[...], preferred_element_type=jnp.float32)
    lg = lg + bp_ref[0:1, :]
    l0 = lg[:, 0:1]
    l1 = lg[:, 1:2]
    mx = jnp.maximum(l0, l1)
    e0 = jnp.exp(l0 - mx)
    e1 = jnp.exp(l1 - mx)
    s = e0 + e1
    lg_ref[...] = jnp.concatenate([l0, l1], axis=1)
    pr_ref[...] = jnp.concatenate([e0 / s, e1 / s], axis=1)


def kernel(x, tables, W1, b1, g1, be1, W2, b2, g2, be2, Wp, bp):
    f32 = jnp.float32
    x0 = x[0]
    xf = x0[:, :2].astype(f32)
    v24 = x0[:, 2:]
    f24 = jnp.arange(2, _F, dtype=jnp.int32)[None, :]
    in_aux = v24 >= _VMAIN
    idx = jnp.where(in_aux,
                    _AUXB + f24 * (_V - _VMAIN) + (v24 - _VMAIN),
                    f24 * _VMAIN + v24).reshape(_NW, _RPW)

    tables_t = jnp.transpose(tables, (0, 2, 1))
    tlin = _relayout_tables(tables_t)
    aux = tables[:, _VMAIN:, :].reshape(-1)
    tlin = lax.dynamic_update_slice(tlin, aux, (_AUXB * _D,))
    tables_flat = tlin.reshape(_F * _V, _D)

    rows = _gather_rows(tables_flat, idx)
    emb_g = rows.reshape(_B, _NG * _D)

    t01 = jnp.pad(tables[:2, 0, :], ((0, 6), (0, 0)))
    W1a = W1[:2 * _D, :]
    W1g = W1[2 * _D:, :]
    b1p = jnp.pad(b1[None, :], ((0, 7), (0, 0)))
    gb1 = jnp.pad(jnp.stack([g1, be1]), ((0, 6), (0, 0)))
    b2p = jnp.pad(b2[None, :], ((0, 7), (0, 0)))
    gb2 = jnp.pad(jnp.stack([g2, be2]), ((0, 6), (0, 0)))
    wpp = jnp.pad(Wp, ((0, 0), (0, 6)))
    bpp = jnp.pad(bp[None, :], ((0, 7), (0, 6)))

    full = lambda shape: pl.BlockSpec(shape, lambda i: (0, 0))
    tile = lambda w: pl.BlockSpec((_TB, w), lambda i: (i, 0))

    h1, st1 = pl.pallas_call(
        _mlp1_body,
        grid=(_NT,),
        in_specs=[tile(_NG * _D), tile(2), full((8, 16)), full((32, _H)),
                  full((_NG * _D, _H)), full((8, _H))],
        out_specs=[tile(_H), full((8, _H))],
        out_shape=[jax.ShapeDtypeStruct((_B, _H), f32),
                   jax.ShapeDtypeStruct((8, _H), f32)],
    )(emb_g, xf, t01, W1a, W1g, b1p)

    h2, st2 = pl.pallas_call(
        _mlp2_body,
        grid=(_NT,),
        in_specs=[tile(_H), full((8, _H)), full((8, _H)), full((_H, _H)),
                  full((8, _H))],
        out_specs=[tile(_H), full((8, _H))],
        out_shape=[jax.ShapeDtypeStruct((_B, _H), f32),
                   jax.ShapeDtypeStruct((8, _H), f32)],
    )(h1, st1, gb1, W2, b2p)

    emb_deep, logit, pred = pl.pallas_call(
        _head_body,
        grid=(_NT,),
        in_specs=[tile(_H), full((8, _H)), full((8, _H)), full((_H, 8)),
                  full((8, 8))],
        out_specs=[tile(_H), tile(2), tile(2)],
        out_shape=[jax.ShapeDtypeStruct((_B, _H), f32),
                   jax.ShapeDtypeStruct((_B, 2), f32),
                   jax.ShapeDtypeStruct((_B, 2), f32)],
    )(h2, st2, gb2, wpp, bpp)

    return (emb_deep, logit, pred)


# speedup vs baseline: 2.6850x; 1.0010x over previous
import functools

import jax
import jax.numpy as jnp
from jax import lax
from jax.experimental import pallas as pl
from jax.experimental.pallas import tpu as pltpu
from jax.experimental.pallas import tpu_sc as plsc

_B = 16384
_F = 26
_V = 100000
_D = 16
_H = 128
_EPS = 1e-5

_NG = _F - 2
_ROWS = _B * _NG
_NC, _NS = 2, 16
_NW = _NC * _NS
_RPW = _ROWS // _NW
_CH = 2048
_NCH = _RPW // _CH

_TB = 1024
_NT = _B // _TB


_VMAIN = (_V // 128) * 128
_AUXB = _F * _VMAIN
_DVLEN = _F * _D * _VMAIN

_K1W = 1408
_K1PF = _VMAIN // _K1W
_K1N = _K1PF * _F
_K1PW = (_K1N + _NW - 1) // _NW

_K2W = 2272
_K2PF = _VMAIN // _K2W
_K2N = _K2PF * _F
_K2PW = (_K2N + _NW - 1) // _NW


def _sc_untile_body(tbl_hbm, out_hbm, staged):
    wid = lax.axis_index("s") * _NC + lax.axis_index("c")

    def body(j, carry):
        g = wid + _NW * j

        @pl.when(g < _K1N)
        def _():
            f = g // _K1PF
            v0 = (g % _K1PF) * _K1W
            pltpu.sync_copy(tbl_hbm.at[f, :, pl.ds(v0, _K1W)], staged)
            for d in range(_D):
                pltpu.sync_copy(
                    staged.at[d, :],
                    out_hbm.at[pl.ds((f * _D + d) * _VMAIN + v0, _K1W)])

        return carry

    lax.fori_loop(0, _K1PW, body, 0)


def _untile_tables(tables_t):
    mesh = plsc.VectorSubcoreMesh(core_axis_name="c", subcore_axis_name="s")
    k = functools.partial(
        pl.kernel,
        mesh=mesh,
        compiler_params=pltpu.CompilerParams(use_tc_tiling_on_sc=True),
        out_type=jax.ShapeDtypeStruct((_DVLEN,), jnp.float32),
        scratch_types=[pltpu.VMEM((_D, _K1W), jnp.float32)],
    )(_sc_untile_body)
    return k(tables_t)


def _sc_transpose_body(dv_hbm, out_hbm, staged, obuf):
    wid = lax.axis_index("s") * _NC + lax.axis_index("c")
    bases = [lax.iota(jnp.int32, 16) * _D + d for d in range(_D)]

    def body(i, carry):
        c = wid + _NW * i

        @pl.when(c < _K2N)
        def _():
            f = c // _K2PF
            v0 = (c % _K2PF) * _K2W
            for d in range(_D):
                pltpu.sync_copy(
                    dv_hbm.at[pl.ds((f * _D + d) * _VMAIN + v0, _K2W)],
                    staged.at[d, :])

            def jbody(j, carry2):
                for d in range(_D):
                    xv = staged[d, pl.ds(j * 16, 16)]
                    plsc.store_scatter(obuf, [bases[d] + j * (16 * _D)], xv)
                return carry2

            lax.fori_loop(0, _K2W // 16, jbody, 0)
            pltpu.sync_copy(
                obuf,
                out_hbm.at[pl.ds((f * _VMAIN + v0) * _D, _K2W * _D)])

        return carry

    lax.fori_loop(0, _K2PW, body, 0)


def _transpose_tables(t_dv):
    mesh = plsc.VectorSubcoreMesh(core_axis_name="c", subcore_axis_name="s")
    k = functools.partial(
        pl.kernel,
        mesh=mesh,
        compiler_params=pltpu.CompilerParams(use_tc_tiling_on_sc=False,
                                             needs_layout_passes=False),
        out_type=jax.ShapeDtypeStruct((_F * _V * _D,), jnp.float32),
        scratch_types=[
            pltpu.VMEM((_D, _K2W), jnp.float32),
            pltpu.VMEM((_K2W * _D,), jnp.float32),
        ],
    )(_sc_transpose_body)
    return k(t_dv)


def _sc_fused_relayout_body(tbl_hbm, out_hbm, st0, st1, ob0, ob1,
                            is0, is1, os0, os1):
    wid = lax.axis_index("s") * _NC + lax.axis_index("c")
    bases = [lax.iota(jnp.int32, 16) * _D + d for d in range(_D)]
    staged = (st0, st1)
    obuf = (ob0, ob1)
    isem = (is0, is1)
    osem = (os0, os1)

    def issue_in(c, b):
        f = c // _K1PF
        v0 = (c % _K1PF) * _K1W
        pltpu.async_copy(tbl_hbm.at[f, :, pl.ds(v0, _K1W)], staged[b], isem[b])

    issue_in(wid, 0)

    def body(t, carry):
        for b in range(2):
            i = t * 2 + b
            c = wid + _NW * i
            cn = c + _NW

            @pl.when(c < _K1N)
            def _():
                pltpu.make_async_copy(
                    tbl_hbm.at[0, :, pl.ds(0, _K1W)], staged[b], isem[b]
                ).wait()

            @pl.when(cn < _K1N)
            def _():
                issue_in(cn, 1 - b)

            @pl.when((i >= 2) & (c < _K1N))
            def _():
                pltpu.make_async_copy(
                    ob0, out_hbm.at[pl.ds(0, _K1W * _D)], osem[b]
                ).wait()

            @pl.when(c < _K1N)
            def _():
                f = c // _K1PF
                v0 = (c % _K1PF) * _K1W

                def jbody(j2, carry2):
                    for u in range(4):
                        j = j2 * 4 + u
                        for d in range(_D):
                            xv = staged[b][d, pl.ds(j * 16, 16)]
                            plsc.store_scatter(
                                obuf[b], [bases[d] + j * (16 * _D)], xv)
                    return carry2

                lax.fori_loop(0, _K1W // 64, jbody, 0)
                pltpu.async_copy(
                    obuf[b],
                    out_hbm.at[pl.ds((f * _VMAIN + v0) * _D, _K1W * _D)],
                    osem[b])

        return carry

    lax.fori_loop(0, (_K1PW + 1) // 2, body, 0)

    for b in range(2):
        pltpu.make_async_copy(
            ob0, out_hbm.at[pl.ds(0, _K1W * _D)], osem[b]
        ).wait()


def _fused_relayout_tables(tables_t):
    mesh = plsc.VectorSubcoreMesh(core_axis_name="c", subcore_axis_name="s")
    k = functools.partial(
        pl.kernel,
        mesh=mesh,
        compiler_params=pltpu.CompilerParams(use_tc_tiling_on_sc=True,
                                             needs_layout_passes=False),
        out_type=jax.ShapeDtypeStruct((_F * _V * _D,), jnp.float32),
        scratch_types=[
            pltpu.VMEM((_D, _K1W), jnp.float32),
            pltpu.VMEM((_D, _K1W), jnp.float32),
            pltpu.VMEM((_K1W * _D,), jnp.float32),
            pltpu.VMEM((_K1W * _D,), jnp.float32),
            pltpu.SemaphoreType.DMA,
            pltpu.SemaphoreType.DMA,
            pltpu.SemaphoreType.DMA,
            pltpu.SemaphoreType.DMA,
        ],
    )(_sc_fused_relayout_body)
    return k(tables_t)


def _relayout_tables(tables_t):
    return _fused_relayout_tables(tables_t)


def _sc_gather_body(tbl_hbm, idx_hbm, out_hbm, idx_v, rows_v, sem):
    wid = lax.axis_index("s") * _NC + lax.axis_index("c")
    pltpu.sync_copy(idx_hbm.at[wid], idx_v)

    def body(c, carry):
        base = wid * _RPW + c * _CH
        pltpu.async_copy(
            tbl_hbm.at[idx_v.at[pl.ds(c * _CH, _CH)]], rows_v, sem
        ).wait()
        pltpu.sync_copy(rows_v, out_hbm.at[pl.ds(base, _CH)])
        return carry

    lax.fori_loop(0, _NCH, body, 0)


def _gather_rows(tables_flat, idx_mat):
    mesh = plsc.VectorSubcoreMesh(core_axis_name="c", subcore_axis_name="s")
    k = functools.partial(
        pl.kernel,
        mesh=mesh,
        compiler_params=pltpu.CompilerParams(use_tc_tiling_on_sc=False),
        out_type=jax.ShapeDtypeStruct((_ROWS, _D), jnp.float32),
        scratch_types=[
            pltpu.VMEM((_RPW,), jnp.int32),
            pltpu.VMEM((_CH, _D), jnp.float32),
            pltpu.SemaphoreType.DMA,
        ],
    )(_sc_gather_body)
    return k(tables_flat, idx_mat)


def _mlp1_body(emb_ref, xf_ref, t01_ref, w1a_ref, w1g_ref, b1_ref,
               h1_ref, st_ref):
    i = pl.program_id(0)
    a0 = xf_ref[:, 0:1] * t01_ref[0:1, :]
    a1 = xf_ref[:, 1:2] * t01_ref[1:2, :]
    h = jnp.dot(emb_ref[...], w1g_ref[...], preferred_element_type=jnp.float32)
    h = h + jnp.dot(a0, w1a_ref[0:16, :], preferred_element_type=jnp.float32)
    h = h + jnp.dot(a1, w1a_ref[16:32, :], preferred_element_type=jnp.float32)
    h = h + b1_ref[0:1, :]
    h1_ref[...] = h

    @pl.when(i == 0)
    def _():
        st_ref[...] = jnp.zeros_like(st_ref)

    st_ref[0:1, :] += jnp.sum(h, axis=0, keepdims=True)
    st_ref[1:2, :] += jnp.sum(h * h, axis=0, keepdims=True)


def _mlp2_body(h1_ref, st1_ref, gb1_ref, w2_ref, b2_ref, h2_ref, st_ref):
    i = pl.program_id(0)
    m = st1_ref[0:1, :] * (1.0 / _B)
    v = st1_ref[1:2, :] * (1.0 / _B) - m * m
    r = lax.rsqrt(v + _EPS)
    h1n = (h1_ref[...] - m) * (r * gb1_ref[0:1, :]) + gb1_ref[1:2, :]
    h2 = jnp.dot(h1n, w2_ref[...], preferred_element_type=jnp.float32)
    h2 = h2 + b2_ref[0:1, :]
    h2_ref[...] = h2

    @pl.when(i == 0)
    def _():
        st_ref[...] = jnp.zeros_like(st_ref)

    st_ref[0:1, :] += jnp.sum(h2, axis=0, keepdims=True)
    st_ref[1:2, :] += jnp.sum(h2 * h2, axis=0, keepdims=True)


def _head_body(h2_ref, st2_ref, gb2_ref, wp_ref, bp_ref,
               ed_ref, lg_ref, pr_ref):
    m = st2_ref[0:1, :] * (1.0 / _B)
    v = st2_ref[1:2, :] * (1.0 / _B) - m * m
    r = lax.rsqrt(v + _EPS)
    h2n = (h2_ref[...] - m) * (r * gb2_ref[0:1, :]) + gb2_ref[1:2, :]
    ed_ref[...] = h2n
    lg = jnp.dot(h2n, wp_ref[...], preferred_element_type=jnp.float32)
    lg = lg + bp_ref[0:1, :]
    l0 = lg[:, 0:1]
    l1 = lg[:, 1:2]
    mx = jnp.maximum(l0, l1)
    e0 = jnp.exp(l0 - mx)
    e1 = jnp.exp(l1 - mx)
    s = e0 + e1
    lg_ref[...] = jnp.concatenate([l0, l1], axis=1)
    pr_ref[...] = jnp.concatenate([e0 / s, e1 / s], axis=1)


def kernel(x, tables, W1, b1, g1, be1, W2, b2, g2, be2, Wp, bp):
    f32 = jnp.float32
    x0 = x[0]
    xf = x0[:, :2].astype(f32)
    v24 = x0[:, 2:]
    f24 = jnp.arange(2, _F, dtype=jnp.int32)[None, :]
    in_aux = v24 >= _VMAIN
    idx = jnp.where(in_aux,
                    _AUXB + f24 * (_V - _VMAIN) + (v24 - _VMAIN),
                    f24 * _VMAIN + v24).reshape(_NW, _RPW)

    tables_t = jnp.transpose(tables, (0, 2, 1))
    tlin = _relayout_tables(tables_t)
    aux = tables[:, _VMAIN:, :].reshape(-1)
    tlin = lax.dynamic_update_slice(tlin, aux, (_AUXB * _D,))
    tables_flat = tlin.reshape(_F * _V, _D)

    rows = _gather_rows(tables_flat, idx)
    emb_g = rows.reshape(_B, _NG * _D)

    t01 = jnp.pad(tables[:2, 0, :], ((0, 6), (0, 0)))
    W1a = W1[:2 * _D, :]
    W1g = W1[2 * _D:, :]
    b1p = jnp.pad(b1[None, :], ((0, 7), (0, 0)))
    gb1 = jnp.pad(jnp.stack([g1, be1]), ((0, 6), (0, 0)))
    b2p = jnp.pad(b2[None, :], ((0, 7), (0, 0)))
    gb2 = jnp.pad(jnp.stack([g2, be2]), ((0, 6), (0, 0)))
    wpp = jnp.pad(Wp, ((0, 0), (0, 6)))
    bpp = jnp.pad(bp[None, :], ((0, 7), (0, 6)))

    full = lambda shape: pl.BlockSpec(shape, lambda i: (0, 0))
    tile = lambda w: pl.BlockSpec((_TB, w), lambda i: (i, 0))

    h1, st1 = pl.pallas_call(
        _mlp1_body,
        grid=(_NT,),
        in_specs=[tile(_NG * _D), tile(2), full((8, 16)), full((32, _H)),
                  full((_NG * _D, _H)), full((8, _H))],
        out_specs=[tile(_H), full((8, _H))],
        out_shape=[jax.ShapeDtypeStruct((_B, _H), f32),
                   jax.ShapeDtypeStruct((8, _H), f32)],
    )(emb_g, xf, t01, W1a, W1g, b1p)

    h2, st2 = pl.pallas_call(
        _mlp2_body,
        grid=(_NT,),
        in_specs=[tile(_H), full((8, _H)), full((8, _H)), full((_H, _H)),
                  full((8, _H))],
        out_specs=[tile(_H), full((8, _H))],
        out_shape=[jax.ShapeDtypeStruct((_B, _H), f32),
                   jax.ShapeDtypeStruct((8, _H), f32)],
    )(h1, st1, gb1, W2, b2p)

    emb_deep, logit, pred = pl.pallas_call(
        _head_body,
        grid=(_NT,),
        in_specs=[tile(_H), full((8, _H)), full((8, _H)), full((_H, 8)),
                  full((8, 8))],
        out_specs=[tile(_H), tile(2), tile(2)],
        out_shape=[jax.ShapeDtypeStruct((_B, _H), f32),
                   jax.ShapeDtypeStruct((_B, 2), f32),
                   jax.ShapeDtypeStruct((_B, 2), f32)],
    )(h2, st2, gb2, wpp, bpp)

    return (emb_deep, logit, pred)


# speedup vs baseline: 2.8093x; 1.0463x over previous
import functools

import jax
import jax.numpy as jnp
from jax import lax
from jax.experimental import pallas as pl
from jax.experimental.pallas import tpu as pltpu
from jax.experimental.pallas import tpu_sc as plsc

_B = 16384
_F = 26
_V = 100000
_D = 16
_H = 128
_EPS = 1e-5

_NG = _F - 2
_ROWS = _B * _NG
_NC, _NS = 2, 16
_NW = _NC * _NS
_RPW = _ROWS // _NW
_CH = 2048
_NCH = _RPW // _CH

_TB = 1024
_NT = _B // _TB


_VMAIN = (_V // 128) * 128
_AUXB = _NG * _VMAIN
_DVLEN = _F * _D * _VMAIN

_K1W = 1408
_K1PF = _VMAIN // _K1W
_K1N = _K1PF * _NG
_K1PW = (_K1N + _NW - 1) // _NW

_K2W = 2272
_K2PF = _VMAIN // _K2W
_K2N = _K2PF * _F
_K2PW = (_K2N + _NW - 1) // _NW


def _sc_untile_body(tbl_hbm, out_hbm, staged):
    wid = lax.axis_index("s") * _NC + lax.axis_index("c")

    def body(j, carry):
        g = wid + _NW * j

        @pl.when(g < _K1N)
        def _():
            f = g // _K1PF
            v0 = (g % _K1PF) * _K1W
            pltpu.sync_copy(tbl_hbm.at[f, :, pl.ds(v0, _K1W)], staged)
            for d in range(_D):
                pltpu.sync_copy(
                    staged.at[d, :],
                    out_hbm.at[pl.ds((f * _D + d) * _VMAIN + v0, _K1W)])

        return carry

    lax.fori_loop(0, _K1PW, body, 0)


def _untile_tables(tables_t):
    mesh = plsc.VectorSubcoreMesh(core_axis_name="c", subcore_axis_name="s")
    k = functools.partial(
        pl.kernel,
        mesh=mesh,
        compiler_params=pltpu.CompilerParams(use_tc_tiling_on_sc=True),
        out_type=jax.ShapeDtypeStruct((_DVLEN,), jnp.float32),
        scratch_types=[pltpu.VMEM((_D, _K1W), jnp.float32)],
    )(_sc_untile_body)
    return k(tables_t)


def _sc_transpose_body(dv_hbm, out_hbm, staged, obuf):
    wid = lax.axis_index("s") * _NC + lax.axis_index("c")
    bases = [lax.iota(jnp.int32, 16) * _D + d for d in range(_D)]

    def body(i, carry):
        c = wid + _NW * i

        @pl.when(c < _K2N)
        def _():
            f = c // _K2PF
            v0 = (c % _K2PF) * _K2W
            for d in range(_D):
                pltpu.sync_copy(
                    dv_hbm.at[pl.ds((f * _D + d) * _VMAIN + v0, _K2W)],
                    staged.at[d, :])

            def jbody(j, carry2):
                for d in range(_D):
                    xv = staged[d, pl.ds(j * 16, 16)]
                    plsc.store_scatter(obuf, [bases[d] + j * (16 * _D)], xv)
                return carry2

            lax.fori_loop(0, _K2W // 16, jbody, 0)
            pltpu.sync_copy(
                obuf,
                out_hbm.at[pl.ds((f * _VMAIN + v0) * _D, _K2W * _D)])

        return carry

    lax.fori_loop(0, _K2PW, body, 0)


def _transpose_tables(t_dv):
    mesh = plsc.VectorSubcoreMesh(core_axis_name="c", subcore_axis_name="s")
    k = functools.partial(
        pl.kernel,
        mesh=mesh,
        compiler_params=pltpu.CompilerParams(use_tc_tiling_on_sc=False,
                                             needs_layout_passes=False),
        out_type=jax.ShapeDtypeStruct((_F * _V * _D,), jnp.float32),
        scratch_types=[
            pltpu.VMEM((_D, _K2W), jnp.float32),
            pltpu.VMEM((_K2W * _D,), jnp.float32),
        ],
    )(_sc_transpose_body)
    return k(t_dv)


def _sc_fused_relayout_body(tbl_hbm, out_hbm, st0, st1, ob0, ob1,
                            is0, is1, os0, os1):
    wid = lax.axis_index("s") * _NC + lax.axis_index("c")
    bases = [lax.iota(jnp.int32, 16) * _D + d for d in range(_D)]
    staged = (st0, st1)
    obuf = (ob0, ob1)
    isem = (is0, is1)
    osem = (os0, os1)

    def issue_in(c, b):
        f = 2 + c // _K1PF
        v0 = (c % _K1PF) * _K1W
        pltpu.async_copy(tbl_hbm.at[f, :, pl.ds(v0, _K1W)], staged[b], isem[b])

    issue_in(wid, 0)

    def body(t, carry):
        for b in range(2):
            i = t * 2 + b
            c = wid + _NW * i
            cn = c + _NW

            @pl.when(c < _K1N)
            def _():
                pltpu.make_async_copy(
                    tbl_hbm.at[0, :, pl.ds(0, _K1W)], staged[b], isem[b]
                ).wait()

            @pl.when(cn < _K1N)
            def _():
                issue_in(cn, 1 - b)

            @pl.when((i >= 2) & (c < _K1N))
            def _():
                pltpu.make_async_copy(
                    ob0, out_hbm.at[pl.ds(0, _K1W * _D)], osem[b]
                ).wait()

            @pl.when(c < _K1N)
            def _():
                f = c // _K1PF
                v0 = (c % _K1PF) * _K1W

                def jbody(j2, carry2):
                    for u in range(4):
                        j = j2 * 4 + u
                        for d in range(_D):
                            xv = staged[b][d, pl.ds(j * 16, 16)]
                            plsc.store_scatter(
                                obuf[b], [bases[d] + j * (16 * _D)], xv)
                    return carry2

                lax.fori_loop(0, _K1W // 64, jbody, 0)
                pltpu.async_copy(
                    obuf[b],
                    out_hbm.at[pl.ds((f * _VMAIN + v0) * _D, _K1W * _D)],
                    osem[b])

        return carry

    lax.fori_loop(0, (_K1PW + 1) // 2, body, 0)

    for b in range(2):
        pltpu.make_async_copy(
            ob0, out_hbm.at[pl.ds(0, _K1W * _D)], osem[b]
        ).wait()


def _fused_relayout_tables(tables_t):
    mesh = plsc.VectorSubcoreMesh(core_axis_name="c", subcore_axis_name="s")
    k = functools.partial(
        pl.kernel,
        mesh=mesh,
        compiler_params=pltpu.CompilerParams(use_tc_tiling_on_sc=True,
                                             needs_layout_passes=False),
        out_type=jax.ShapeDtypeStruct((_NG * _V * _D,), jnp.float32),
        scratch_types=[
            pltpu.VMEM((_D, _K1W), jnp.float32),
            pltpu.VMEM((_D, _K1W), jnp.float32),
            pltpu.VMEM((_K1W * _D,), jnp.float32),
            pltpu.VMEM((_K1W * _D,), jnp.float32),
            pltpu.SemaphoreType.DMA,
            pltpu.SemaphoreType.DMA,
            pltpu.SemaphoreType.DMA,
            pltpu.SemaphoreType.DMA,
        ],
    )(_sc_fused_relayout_body)
    return k(tables_t)


def _relayout_tables(tables_t):
    return _fused_relayout_tables(tables_t)


def _sc_gather_body(tbl_hbm, idx_hbm, out_hbm, idx_v, rows_v, sem):
    wid = lax.axis_index("s") * _NC + lax.axis_index("c")
    pltpu.sync_copy(idx_hbm.at[wid], idx_v)

    def body(c, carry):
        base = wid * _RPW + c * _CH
        pltpu.async_copy(
            tbl_hbm.at[idx_v.at[pl.ds(c * _CH, _CH)]], rows_v, sem
        ).wait()
        pltpu.sync_copy(rows_v, out_hbm.at[pl.ds(base, _CH)])
        return carry

    lax.fori_loop(0, _NCH, body, 0)


def _gather_rows(tables_flat, idx_mat):
    mesh = plsc.VectorSubcoreMesh(core_axis_name="c", subcore_axis_name="s")
    k = functools.partial(
        pl.kernel,
        mesh=mesh,
        compiler_params=pltpu.CompilerParams(use_tc_tiling_on_sc=False),
        out_type=jax.ShapeDtypeStruct((_ROWS, _D), jnp.float32),
        scratch_types=[
            pltpu.VMEM((_RPW,), jnp.int32),
            pltpu.VMEM((_CH, _D), jnp.float32),
            pltpu.SemaphoreType.DMA,
        ],
    )(_sc_gather_body)
    return k(tables_flat, idx_mat)


def _mlp1_body(emb_ref, xf_ref, t01_ref, w1a_ref, w1g_ref, b1_ref,
               h1_ref, st_ref):
    i = pl.program_id(0)
    a0 = xf_ref[:, 0:1] * t01_ref[0:1, :]
    a1 = xf_ref[:, 1:2] * t01_ref[1:2, :]
    h = jnp.dot(emb_ref[...], w1g_ref[...], preferred_element_type=jnp.float32)
    h = h + jnp.dot(a0, w1a_ref[0:16, :], preferred_element_type=jnp.float32)
    h = h + jnp.dot(a1, w1a_ref[16:32, :], preferred_element_type=jnp.float32)
    h = h + b1_ref[0:1, :]
    h1_ref[...] = h

    @pl.when(i == 0)
    def _():
        st_ref[...] = jnp.zeros_like(st_ref)

    st_ref[0:1, :] += jnp.sum(h, axis=0, keepdims=True)
    st_ref[1:2, :] += jnp.sum(h * h, axis=0, keepdims=True)


def _mlp2_body(h1_ref, st1_ref, gb1_ref, w2_ref, b2_ref, h2_ref, st_ref):
    i = pl.program_id(0)
    m = st1_ref[0:1, :] * (1.0 / _B)
    v = st1_ref[1:2, :] * (1.0 / _B) - m * m
    r = lax.rsqrt(v + _EPS)
    h1n = (h1_ref[...] - m) * (r * gb1_ref[0:1, :]) + gb1_ref[1:2, :]
    h2 = jnp.dot(h1n, w2_ref[...], preferred_element_type=jnp.float32)
    h2 = h2 + b2_ref[0:1, :]
    h2_ref[...] = h2

    @pl.when(i == 0)
    def _():
        st_ref[...] = jnp.zeros_like(st_ref)

    st_ref[0:1, :] += jnp.sum(h2, axis=0, keepdims=True)
    st_ref[1:2, :] += jnp.sum(h2 * h2, axis=0, keepdims=True)


def _head_body(h2_ref, st2_ref, gb2_ref, wp_ref, bp_ref,
               ed_ref, lg_ref, pr_ref):
    m = st2_ref[0:1, :] * (1.0 / _B)
    v = st2_ref[1:2, :] * (1.0 / _B) - m * m
    r = lax.rsqrt(v + _EPS)
    h2n = (h2_ref[...] - m) * (r * gb2_ref[0:1, :]) + gb2_ref[1:2, :]
    ed_ref[...] = h2n
    lg = jnp.dot(h2n, wp_ref[...], preferred_element_type=jnp.float32)
    lg = lg + bp_ref[0:1, :]
    l0 = lg[:, 0:1]
    l1 = lg[:, 1:2]
    mx = jnp.maximum(l0, l1)
    e0 = jnp.exp(l0 - mx)
    e1 = jnp.exp(l1 - mx)
    s = e0 + e1
    lg_ref[...] = jnp.concatenate([l0, l1], axis=1)
    pr_ref[...] = jnp.concatenate([e0 / s, e1 / s], axis=1)


def kernel(x, tables, W1, b1, g1, be1, W2, b2, g2, be2, Wp, bp):
    f32 = jnp.float32
    x0 = x[0]
    xf = x0[:, :2].astype(f32)
    v24 = x0[:, 2:]
    g24 = jnp.arange(_NG, dtype=jnp.int32)[None, :]
    in_aux = v24 >= _VMAIN
    idx = jnp.where(in_aux,
                    _AUXB + g24 * (_V - _VMAIN) + (v24 - _VMAIN),
                    g24 * _VMAIN + v24).reshape(_NW, _RPW)

    tables_t = jnp.transpose(tables, (0, 2, 1))
    tlin = _relayout_tables(tables_t)
    aux = tables[2:, _VMAIN:, :].reshape(-1)
    tlin = lax.dynamic_update_slice(tlin, aux, (_AUXB * _D,))
    tables_flat = tlin.reshape(_NG * _V, _D)

    rows = _gather_rows(tables_flat, idx)
    emb_g = rows.reshape(_B, _NG * _D)

    t01 = jnp.pad(tables[:2, 0, :], ((0, 6), (0, 0)))
    W1a = W1[:2 * _D, :]
    W1g = W1[2 * _D:, :]
    b1p = jnp.pad(b1[None, :], ((0, 7), (0, 0)))
    gb1 = jnp.pad(jnp.stack([g1, be1]), ((0, 6), (0, 0)))
    b2p = jnp.pad(b2[None, :], ((0, 7), (0, 0)))
    gb2 = jnp.pad(jnp.stack([g2, be2]), ((0, 6), (0, 0)))
    wpp = jnp.pad(Wp, ((0, 0), (0, 6)))
    bpp = jnp.pad(bp[None, :], ((0, 7), (0, 6)))

    full = lambda shape: pl.BlockSpec(shape, lambda i: (0, 0))
    tile = lambda w: pl.BlockSpec((_TB, w), lambda i: (i, 0))

    h1, st1 = pl.pallas_call(
        _mlp1_body,
        grid=(_NT,),
        in_specs=[tile(_NG * _D), tile(2), full((8, 16)), full((32, _H)),
                  full((_NG * _D, _H)), full((8, _H))],
        out_specs=[tile(_H), full((8, _H))],
        out_shape=[jax.ShapeDtypeStruct((_B, _H), f32),
                   jax.ShapeDtypeStruct((8, _H), f32)],
    )(emb_g, xf, t01, W1a, W1g, b1p)

    h2, st2 = pl.pallas_call(
        _mlp2_body,
        grid=(_NT,),
        in_specs=[tile(_H), full((8, _H)), full((8, _H)), full((_H, _H)),
                  full((8, _H))],
        out_specs=[tile(_H), full((8, _H))],
        out_shape=[jax.ShapeDtypeStruct((_B, _H), f32),
                   jax.ShapeDtypeStruct((8, _H), f32)],
    )(h1, st1, gb1, W2, b2p)

    emb_deep, logit, pred = pl.pallas_call(
        _head_body,
        grid=(_NT,),
        in_specs=[tile(_H), full((8, _H)), full((8, _H)), full((_H, 8)),
                  full((8, 8))],
        out_specs=[tile(_H), tile(2), tile(2)],
        out_shape=[jax.ShapeDtypeStruct((_B, _H), f32),
                   jax.ShapeDtypeStruct((_B, 2), f32),
                   jax.ShapeDtypeStruct((_B, 2), f32)],
    )(h2, st2, gb2, wpp, bpp)

    return (emb_deep, logit, pred)
